# baseline (speedup 1.0000x reference)
import functools

import jax
import jax.numpy as jnp
from jax import lax
from jax.experimental import pallas as pl
from jax.experimental.pallas import tpu as pltpu
from jax.experimental.pallas import tpu_sc as plsc

_B = 1024
_V = 100000
_CH = 50000
_UN = 5
_NV = _CH // 16


def _sc_count(pred_flat, flatrep, tgtrep):
    info = plsc.get_sparse_core_info()
    nc, ns, lanes = info.num_cores, info.num_subcores, info.num_lanes
    nw = nc * ns
    bpw = _B // nw
    seg = bpw * lanes

    mesh = plsc.VectorSubcoreMesh(core_axis_name="c", subcore_axis_name="s")

    @functools.partial(
        pl.kernel,
        mesh=mesh,
        out_type=(
            jax.ShapeDtypeStruct((_B * lanes,), jnp.int32),
            jax.ShapeDtypeStruct((_B * lanes,), jnp.float32),
        ),
        scratch_types=[
            pltpu.VMEM((seg,), jnp.int32),
            pltpu.VMEM((seg,), jnp.int32),
            pltpu.VMEM((seg,), jnp.float32),
            pltpu.VMEM((seg,), jnp.int32),
            pltpu.VMEM((_CH,), jnp.float32),
            pltpu.VMEM((_CH,), jnp.float32),
            pltpu.SemaphoreType.DMA,
            pltpu.SemaphoreType.DMA,
            pltpu.SemaphoreType.DMA,
        ],
    )
    def k(predf_hbm, flatrep_hbm, tgtrep_hbm, parts_hbm, tvals_hbm,
          fidx_v, tgtspl_v, tspl_v, resacc_v, buf0, buf1,
          sem_g, sem0, sem1):
        wid = lax.axis_index("s") * nc + lax.axis_index("c")
        base = wid * bpw

        pltpu.sync_copy(tgtrep_hbm.at[pl.ds(wid * seg, seg)], tgtspl_v)
        pltpu.sync_copy(flatrep_hbm.at[pl.ds(wid * seg, seg)], fidx_v)
        for q in range(seg // 128):
            pltpu.async_copy(
                predf_hbm.at[fidx_v.at[pl.ds(q * 128, 128)]],
                tspl_v.at[pl.ds(q * 128, 128)], sem_g).wait()

        def start(row, h, buf, sem):
            pltpu.async_copy(
                predf_hbm.at[pl.ds((base + row) * _V + h * _CH, _CH)],
                buf, sem)

        def wait(buf, sem):
            pltpu.make_async_copy(
                predf_hbm.at[pl.ds(0, _CH)], buf, sem).wait()

        def compute(row, h, buf, acc):
            t_spl = tspl_v[pl.ds(row * lanes, lanes)]
            tg_spl = tgtspl_v[pl.ds(row * lanes, lanes)]
            col0 = jnp.full((lanes,), h * _CH, jnp.int32) + lax.iota(
                jnp.int32, lanes)

            def inner(kk, c):
                a, col = c
                for s in range(_UN):
                    v = buf[pl.ds((kk * _UN + s) * lanes, lanes)]
                    m = (v > t_spl) | ((v == t_spl) & (col < tg_spl))
                    a = a + jnp.where(m, 1, 0).astype(jnp.int32)
                    col = col + lanes
                return a, col

            acc, _ = lax.fori_loop(0, _NV // _UN, inner, (acc, col0))
            if h == 1:
                resacc_v[pl.ds(row * lanes, lanes)] = acc
                acc = jnp.zeros((lanes,), jnp.int32)
            return acc

        start(0, 0, buf0, sem0)
        start(0, 1, buf1, sem1)

        def outer(i, acc):
            wait(buf0, sem0)
            acc = compute(i, 0, buf0, acc)
            start(i + 1, 0, buf0, sem0)
            wait(buf1, sem1)
            acc = compute(i, 1, buf1, acc)
            start(i + 1, 1, buf1, sem1)
            return acc

        acc = jnp.zeros((lanes,), jnp.int32)
        acc = lax.fori_loop(0, bpw - 1, outer, acc)
        wait(buf0, sem0)
        acc = compute(bpw - 1, 0, buf0, acc)
        wait(buf1, sem1)
        compute(bpw - 1, 1, buf1, acc)

        pltpu.sync_copy(resacc_v, parts_hbm.at[pl.ds(wid * seg, seg)])
        pltpu.sync_copy(tspl_v, tvals_hbm.at[pl.ds(wid * seg, seg)])

    return k(pred_flat, flatrep, tgtrep)


def _finish_body(p_ref, t_ref, out_ref):
    rank = jnp.sum(p_ref[...], axis=1, keepdims=True)
    pos = t_ref[...][:, :1] > 0.0
    s1 = jnp.sum(((rank == 0) & pos).astype(jnp.float32))
    s5 = jnp.sum(((rank < 5) & pos).astype(jnp.float32))
    cc = jnp.sum((rank < 5).astype(jnp.float32))
    l2 = lax.broadcasted_iota(jnp.int32, (1, 128), 1)
    out_ref[...] = jnp.where(l2 == 0, s1,
                             jnp.where(l2 == 1, s5,
                                       jnp.where(l2 == 2, cc, 0.0)))


def _finish(parts, tvals):
    return pl.pallas_call(
        _finish_body,
        out_shape=jax.ShapeDtypeStruct((1, 128), jnp.float32),
    )(parts, tvals)


def kernel(pred, target):
    b, v = pred.shape
    tgt = target.astype(jnp.int32)
    flat = jnp.arange(b, dtype=jnp.int32) * v + tgt
    flatrep = jnp.repeat(flat, 16)
    tgtrep = jnp.repeat(tgt, 16)
    parts, tvals = _sc_count(pred.reshape(-1), flatrep, tgtrep)
    out = _finish(parts.reshape(b, 16), tvals.reshape(b, 16))
    res = jnp.stack([out[0, 0], out[0, 1]]) * (100.0 / b)
    return (res, out[0, 2].astype(jnp.int32))


# speedup vs baseline: 1.1293x; 1.1293x over previous
import functools

import jax
import jax.numpy as jnp
from jax import lax
from jax.experimental import pallas as pl
from jax.experimental.pallas import tpu as pltpu
from jax.experimental.pallas import tpu_sc as plsc

_B = 1024
_V = 100000
_VS = 99968
_CK = 1408
_NCK = _VS // _CK
_UN = 8


def _sc_count(pred, tgtp, tail):
    info = plsc.get_sparse_core_info()
    nc, ns, lanes = info.num_cores, info.num_subcores, info.num_lanes
    nw = nc * ns
    bpw = _B // nw
    ngrp = bpw // 8
    seg = bpw * lanes

    mesh = plsc.VectorSubcoreMesh(core_axis_name="c", subcore_axis_name="s")

    @functools.partial(
        pl.kernel,
        mesh=mesh,
        out_type=(
            jax.ShapeDtypeStruct((_B * lanes,), jnp.int32),
            jax.ShapeDtypeStruct((_B * lanes,), jnp.float32),
        ),
        scratch_types=[
            pltpu.VMEM((8, _CK), jnp.float32),
            pltpu.VMEM((8, _CK), jnp.float32),
            pltpu.VMEM((8, _CK), jnp.float32),
            pltpu.VMEM((8, _CK), jnp.float32),
            pltpu.VMEM((8, 8, 128), jnp.float32),
            pltpu.VMEM((8, 128), jnp.float32),
            pltpu.VMEM((48,), jnp.int32),
            pltpu.VMEM((seg,), jnp.float32),
            pltpu.VMEM((seg,), jnp.int32),
            pltpu.VMEM((seg,), jnp.int32),
            pltpu.SemaphoreType.DMA,
            pltpu.SemaphoreType.DMA,
            pltpu.SemaphoreType.DMA,
            pltpu.SemaphoreType.DMA,
            pltpu.SemaphoreType.DMA,
        ],
    )
    def k(pred_hbm, tgtp_hbm, tail_hbm, parts_hbm, tvals_hbm,
          b0, b1, b2, b3, winb, tailw, tgt_v, tband_v, tgspl_v, resacc_v,
          semw, s0, s1, s2, s3):
        wid = lax.axis_index("s") * nc + lax.axis_index("c")
        base = wid * bpw
        bufs = (b0, b1, b2, b3)
        sems = (s0, s1, s2, s3)

        pltpu.sync_copy(tgtp_hbm.at[pl.ds(base, 48)], tgt_v)

        def start(u, buf, sem, g):
            pltpu.async_copy(
                pred_hbm.at[pl.ds(base + g * 8, 8), pl.ds(u * _CK, _CK)],
                buf, sem)

        def wait(buf, sem):
            pltpu.make_async_copy(
                pred_hbm.at[pl.ds(0, 8), pl.ds(0, _CK)], buf, sem).wait()

        def comp(u, buf, g):
            cb = u * _CK

            def rows(r, _):
                off = (g * 8 + r) * lanes
                t_spl = tband_v[pl.ds(off, lanes)]
                tg_spl = tgspl_v[pl.ds(off, lanes)]
                acc0 = resacc_v[pl.ds(off, lanes)]
                col0 = jnp.full((lanes,), cb, jnp.int32) + lax.iota(
                    jnp.int32, lanes)

                def innerk(kk, c):
                    a, col = c
                    for s in range(_UN):
                        v = buf[r, pl.ds((kk * _UN + s) * lanes, lanes)]
                        m = (v > t_spl) | ((v == t_spl) & (col < tg_spl))
                        a = a + jnp.where(m, 1, 0)
                        col = col + lanes
                    return a, col

                acc, _ = lax.fori_loop(0, _CK // lanes // _UN, innerk,
                                       (acc0, col0))
                resacc_v[pl.ds(off, lanes)] = acc
                return 0

            lax.fori_loop(0, 8, rows, 0)

        def group(g, _):
            tvec = tgt_v[pl.ds(g * 8, lanes)]
            for r in range(8):
                tg_r = tvec[r]
                ct = (jnp.minimum(tg_r, _VS - 1) // 128) * 128
                pltpu.async_copy(
                    pred_hbm.at[pl.ds(base + g * 8, 8), pl.ds(ct, 128)],
                    winb.at[r], semw)
            pltpu.async_copy(
                tail_hbm.at[pl.ds(base + g * 8, 8), pl.ds(0, 128)],
                tailw, semw)
            for bb in range(4):
                start(bb, bufs[bb], sems[bb], g)
            for r in range(8):
                pltpu.make_async_copy(
                    pred_hbm.at[pl.ds(0, 8), pl.ds(0, 128)],
                    winb.at[r], semw).wait()
            pltpu.make_async_copy(
                tail_hbm.at[pl.ds(0, 8), pl.ds(0, 128)], tailw, semw).wait()

            neg = jnp.float32(-jnp.inf)
            ii = lax.iota(jnp.int32, lanes)
            for r in range(8):
                tg_r = tvec[r]
                off = (g * 8 + r) * lanes
                ct = (jnp.minimum(tg_r, _VS - 1) // 128) * 128
                pm = jnp.clip(tg_r - ct, 0, 127)
                pt = jnp.clip(tg_r - _VS, 0, 127)
                vm = winb[r, r, pl.ds((pm // lanes) * lanes, lanes)]
                cm = jnp.where(ii == (pm % lanes), vm, neg)
                vt = tailw[r, pl.ds((pt // lanes) * lanes, lanes)]
                ctl = jnp.where(ii == (pt % lanes), vt, neg)
                x = jnp.where(tg_r < _VS, cm, ctl)
                for sh in (1, 2, 4, 8):
                    x = jnp.maximum(x, jnp.take(x, ii ^ sh))
                tband_v[pl.ds(off, lanes)] = x
                tgspl_v[pl.ds(off, lanes)] = jnp.full((lanes,), tg_r,
                                                      jnp.int32)
                resacc_v[pl.ds(off, lanes)] = jnp.zeros((lanes,), jnp.int32)

            for r in range(8):
                off = (g * 8 + r) * lanes
                t_spl = tband_v[pl.ds(off, lanes)]
                tg_spl = tgspl_v[pl.ds(off, lanes)]
                acc = resacc_v[pl.ds(off, lanes)]
                for s in range(8):
                    v = tailw[r, pl.ds(s * lanes, lanes)]
                    col = jnp.full((lanes,), _VS + s * lanes, jnp.int32) + ii
                    m = (v > t_spl) | ((v == t_spl) & (col < tg_spl))
                    acc = acc + jnp.where(m, 1, 0)
                resacc_v[pl.ds(off, lanes)] = acc

            def quad(q, _):
                for bb in range(4):
                    u = q * 4 + bb
                    wait(bufs[bb], sems[bb])
                    comp(u, bufs[bb], g)
                    start(u + 4, bufs[bb], sems[bb], g)
                return 0

            lax.fori_loop(0, (_NCK - 7) // 4, quad, 0)
            for u0 in range(_NCK - 7, _NCK):
                bb = u0 % 4
                wait(bufs[bb], sems[bb])
                comp(jnp.int32(u0), bufs[bb], g)
                if u0 + 4 < _NCK:
                    start(u0 + 4, bufs[bb], sems[bb], g)
            return 0

        lax.fori_loop(0, ngrp, group, 0)

        pltpu.sync_copy(resacc_v, parts_hbm.at[pl.ds(wid * seg, seg)])
        pltpu.sync_copy(tband_v, tvals_hbm.at[pl.ds(wid * seg, seg)])

    return k(pred, tgtp, tail)


def _finish_body(p_ref, t_ref, out_ref):
    rank = jnp.sum(p_ref[...], axis=1, keepdims=True)
    pos = t_ref[...][:, :1] > 0.0
    s1 = jnp.sum(((rank == 0) & pos).astype(jnp.float32))
    s5 = jnp.sum(((rank < 5) & pos).astype(jnp.float32))
    cc = jnp.sum((rank < 5).astype(jnp.float32))
    l2 = lax.broadcasted_iota(jnp.int32, (1, 128), 1)
    out_ref[...] = jnp.where(l2 == 0, s1,
                             jnp.where(l2 == 1, s5,
                                       jnp.where(l2 == 2, cc, 0.0)))


def _finish(parts, tvals):
    return pl.pallas_call(
        _finish_body,
        out_shape=jax.ShapeDtypeStruct((1, 128), jnp.float32),
    )(parts, tvals)


def kernel(pred, target):
    b, v = pred.shape
    tgt = target.astype(jnp.int32)
    tgtp = jnp.concatenate([tgt, jnp.zeros((16,), jnp.int32)])
    tail = jnp.pad(pred[:, _VS:], ((0, 0), (0, 128 - (v - _VS))),
                   constant_values=-jnp.inf)
    parts, tvals = _sc_count(pred, tgtp, tail)
    out = _finish(parts.reshape(b, 16), tvals.reshape(b, 16))
    res = jnp.stack([out[0, 0], out[0, 1]]) * (100.0 / b)
    return (res, out[0, 2].astype(jnp.int32))


# speedup vs baseline: 1.4550x; 1.2884x over previous
import functools

import jax
import jax.numpy as jnp
from jax import lax
from jax.experimental import pallas as pl
from jax.experimental.pallas import tpu as pltpu
from jax.experimental.pallas import tpu_sc as plsc

_B = 1024
_V = 100000
_RT = 512
_VS = 99968
_CK = 1408
_NCK = _VS // _CK
_UN = 8
_BR = 256
_BC = 8192
_CB = -(-_V // _BC)


def _sc_kernel(pred, tgtp, tail, row0, nrows, stream):
    info = plsc.get_sparse_core_info()
    nc, ns, lanes = info.num_cores, info.num_subcores, info.num_lanes
    nw = nc * ns
    bpw = nrows // nw
    ngrp = bpw // 8
    seg = bpw * lanes

    mesh = plsc.VectorSubcoreMesh(core_axis_name="c", subcore_axis_name="s")

    @functools.partial(
        pl.kernel,
        mesh=mesh,
        out_type=(
            jax.ShapeDtypeStruct((nrows * lanes,), jnp.int32),
            jax.ShapeDtypeStruct((nrows * lanes,), jnp.float32),
        ),
        scratch_types=[
            pltpu.VMEM((8, _CK), jnp.float32),
            pltpu.VMEM((8, _CK), jnp.float32),
            pltpu.VMEM((8, _CK), jnp.float32),
            pltpu.VMEM((8, _CK), jnp.float32),
            pltpu.VMEM((8, 8, 128), jnp.float32),
            pltpu.VMEM((8, 128), jnp.float32),
            pltpu.VMEM((48,), jnp.int32),
            pltpu.VMEM((seg,), jnp.float32),
            pltpu.VMEM((seg,), jnp.int32),
            pltpu.VMEM((seg,), jnp.int32),
            pltpu.SemaphoreType.DMA,
            pltpu.SemaphoreType.DMA,
            pltpu.SemaphoreType.DMA,
            pltpu.SemaphoreType.DMA,
            pltpu.SemaphoreType.DMA,
        ],
    )
    def k(pred_hbm, tgtp_hbm, tail_hbm, parts_hbm, tvals_hbm,
          b0, b1, b2, b3, winb, tailw, tgt_v, tband_v, tgspl_v, resacc_v,
          semw, s0, s1, s2, s3):
        wid = lax.axis_index("s") * nc + lax.axis_index("c")
        base = row0 + wid * bpw
        bufs = (b0, b1, b2, b3)
        sems = (s0, s1, s2, s3)

        pltpu.sync_copy(tgtp_hbm.at[pl.ds(base, 48)], tgt_v)

        def start(u, buf, sem, g):
            pltpu.async_copy(
                pred_hbm.at[pl.ds(base + g * 8, 8), pl.ds(u * _CK, _CK)],
                buf, sem)

        def wait(buf, sem):
            pltpu.make_async_copy(
                pred_hbm.at[pl.ds(0, 8), pl.ds(0, _CK)], buf, sem).wait()

        def comp(u, buf, g):
            cb = u * _CK

            def rows(r, _):
                off = (g * 8 + r) * lanes
                t_spl = tband_v[pl.ds(off, lanes)]
                tg_spl = tgspl_v[pl.ds(off, lanes)]
                acc0 = resacc_v[pl.ds(off, lanes)]
                col0 = jnp.full((lanes,), cb, jnp.int32) + lax.iota(
                    jnp.int32, lanes)

                def innerk(kk, c):
                    a, col = c
                    for s in range(_UN):
                        v = buf[r, pl.ds((kk * _UN + s) * lanes, lanes)]
                        m = (v > t_spl) | ((v == t_spl) & (col < tg_spl))
                        a = a + jnp.where(m, 1, 0)
                        col = col + lanes
                    return a, col

                acc, _ = lax.fori_loop(0, _CK // lanes // _UN, innerk,
                                       (acc0, col0))
                resacc_v[pl.ds(off, lanes)] = acc
                return 0

            lax.fori_loop(0, 8, rows, 0)

        def group(g, _):
            tvec = tgt_v[pl.ds(g * 8, lanes)]
            for r in range(8):
                tg_r = tvec[r]
                ct = (jnp.minimum(tg_r, _VS - 1) // 128) * 128
                pltpu.async_copy(
                    pred_hbm.at[pl.ds(base + g * 8, 8), pl.ds(ct, 128)],
                    winb.at[r], semw)
            pltpu.async_copy(
                tail_hbm.at[pl.ds(base + g * 8, 8), pl.ds(0, 128)],
                tailw, semw)
            if stream:
                for bb in range(4):
                    start(bb, bufs[bb], sems[bb], g)
            for r in range(8):
                pltpu.make_async_copy(
                    pred_hbm.at[pl.ds(0, 8), pl.ds(0, 128)],
                    winb.at[r], semw).wait()
            pltpu.make_async_copy(
                tail_hbm.at[pl.ds(0, 8), pl.ds(0, 128)], tailw, semw).wait()

            neg = jnp.float32(-jnp.inf)
            ii = lax.iota(jnp.int32, lanes)
            for r in range(8):
                tg_r = tvec[r]
                off = (g * 8 + r) * lanes
                ct = (jnp.minimum(tg_r, _VS - 1) // 128) * 128
                pm = jnp.clip(tg_r - ct, 0, 127)
                pt = jnp.clip(tg_r - _VS, 0, 127)
                vm = winb[r, r, pl.ds((pm // lanes) * lanes, lanes)]
                cm = jnp.where(ii == (pm % lanes), vm, neg)
                vt = tailw[r, pl.ds((pt // lanes) * lanes, lanes)]
                ctl = jnp.where(ii == (pt % lanes), vt, neg)
                x = jnp.where(tg_r < _VS, cm, ctl)
                for sh in (1, 2, 4, 8):
                    x = jnp.maximum(x, jnp.take(x, ii ^ sh))
                tband_v[pl.ds(off, lanes)] = x
                tgspl_v[pl.ds(off, lanes)] = jnp.full((lanes,), tg_r,
                                                      jnp.int32)
                resacc_v[pl.ds(off, lanes)] = jnp.zeros((lanes,), jnp.int32)

            if stream:
                for r in range(8):
                    off = (g * 8 + r) * lanes
                    t_spl = tband_v[pl.ds(off, lanes)]
                    tg_spl = tgspl_v[pl.ds(off, lanes)]
                    acc = resacc_v[pl.ds(off, lanes)]
                    for s in range(8):
                        v = tailw[r, pl.ds(s * lanes, lanes)]
                        col = jnp.full((lanes,), _VS + s * lanes,
                                       jnp.int32) + ii
                        m = (v > t_spl) | ((v == t_spl) & (col < tg_spl))
                        acc = acc + jnp.where(m, 1, 0)
                    resacc_v[pl.ds(off, lanes)] = acc

                def quad(q, _):
                    for bb in range(4):
                        u = q * 4 + bb
                        wait(bufs[bb], sems[bb])
                        comp(u, bufs[bb], g)
                        start(u + 4, bufs[bb], sems[bb], g)
                    return 0

                lax.fori_loop(0, (_NCK - 7) // 4, quad, 0)
                for u0 in range(_NCK - 7, _NCK):
                    bb = u0 % 4
                    wait(bufs[bb], sems[bb])
                    comp(jnp.int32(u0), bufs[bb], g)
                    if u0 + 4 < _NCK:
                        start(u0 + 4, bufs[bb], sems[bb], g)
            return 0

        lax.fori_loop(0, ngrp, group, 0)

        pltpu.sync_copy(resacc_v, parts_hbm.at[pl.ds(wid * seg, seg)])
        pltpu.sync_copy(tband_v, tvals_hbm.at[pl.ds(wid * seg, seg)])

    return k(pred, tgtp, tail)


def _tc_body(pred_ref, t_ref, tgt_ref, out_ref, acc_ref):
    r = pl.program_id(0)
    c = pl.program_id(1)

    @pl.when(c == 0)
    def _init():
        acc_ref[...] = jnp.zeros_like(acc_ref)

    p = pred_ref[...]
    t = t_ref[...]
    tg = tgt_ref[...]
    col = c * _BC + lax.broadcasted_iota(jnp.int32, (_BR, _BC), 1)
    hit = ((p > t) & (col < _V)) | ((p == t) & (col < tg))
    acc_ref[...] += jnp.sum(hit.astype(jnp.int32), axis=1, keepdims=True)

    @pl.when(c == _CB - 1)
    def _finish():
        rank = acc_ref[...]
        pos = t > 0.0
        s1 = jnp.sum(((rank == 0) & pos).astype(jnp.float32))
        s5 = jnp.sum(((rank < 5) & pos).astype(jnp.float32))
        cc = jnp.sum((rank < 5).astype(jnp.float32))
        lane = lax.broadcasted_iota(jnp.int32, (1, 128), 1)
        vec = jnp.where(lane == 0, s1,
                        jnp.where(lane == 1, s5,
                                  jnp.where(lane == 2, cc, 0.0)))

        @pl.when(r == 0)
        def _():
            out_ref[...] = vec

        @pl.when(r > 0)
        def _():
            out_ref[...] += vec


def _tc_count(pred, t2, tgt2):
    return pl.pallas_call(
        _tc_body,
        grid=(_RT // _BR, _CB),
        in_specs=[
            pl.BlockSpec((_BR, _BC), lambda r, c: (r, c)),
            pl.BlockSpec((_BR, 1), lambda r, c: (r, 0)),
            pl.BlockSpec((_BR, 1), lambda r, c: (r, 0)),
        ],
        out_specs=pl.BlockSpec((1, 128), lambda r, c: (0, 0)),
        out_shape=jax.ShapeDtypeStruct((1, 128), jnp.float32),
        scratch_shapes=[pltpu.VMEM((_BR, 1), jnp.int32)],
    )(pred, t2, tgt2)


def _finish_body(p_ref, t_ref, tc_ref, out_ref):
    rank = jnp.sum(p_ref[...], axis=1, keepdims=True)
    pos = t_ref[...][:, :1] > 0.0
    s1 = jnp.sum(((rank == 0) & pos).astype(jnp.float32))
    s5 = jnp.sum(((rank < 5) & pos).astype(jnp.float32))
    cc = jnp.sum((rank < 5).astype(jnp.float32))
    l2 = lax.broadcasted_iota(jnp.int32, (1, 128), 1)
    out_ref[...] = tc_ref[...] + jnp.where(
        l2 == 0, s1, jnp.where(l2 == 1, s5, jnp.where(l2 == 2, cc, 0.0)))


def _finish(parts, tvals, tcpart):
    return pl.pallas_call(
        _finish_body,
        out_shape=jax.ShapeDtypeStruct((1, 128), jnp.float32),
    )(parts, tvals, tcpart)


def kernel(pred, target):
    b, v = pred.shape
    tgt = target.astype(jnp.int32)
    tgtp = jnp.concatenate([tgt, jnp.zeros((32,), jnp.int32)])
    tail = jnp.pad(pred[:, _VS:], ((0, 0), (0, 128 - (v - _VS))),
                   constant_values=-jnp.inf)
    _, tvals_tc = _sc_kernel(pred, tgtp, tail, 0, _RT, stream=False)
    t2 = tvals_tc.reshape(_RT, 16)[:, :1]
    tgt2 = tgt[:_RT].reshape(_RT, 1)
    tcpart = _tc_count(pred, t2, tgt2)
    parts, tvals = _sc_kernel(pred, tgtp, tail, _RT, b - _RT, stream=True)
    out = _finish(parts.reshape(b - _RT, 16), tvals.reshape(b - _RT, 16),
                  tcpart)
    res = jnp.stack([out[0, 0], out[0, 1]]) * (100.0 / b)
    return (res, out[0, 2].astype(jnp.int32))
